# Initial kernel scaffold; baseline (speedup 1.0000x reference)
import jax
import jax.numpy as jnp
from jax.experimental import pallas as pl


def kernel(x, edge_index, W, att_src, att_dst, bias):
    raise NotImplementedError("write your pallas kernel here")


# speedup vs baseline: 1.4846x; 1.4846x over previous
import functools
import jax
import jax.numpy as jnp
from jax.experimental import pallas as pl
from jax.experimental.pallas import tpu as pltpu

NEG_SLOPE = 0.2
CH = 8192


def _proj_body(x_ref, w_ref, asf_ref, adf_ref, g_ref, h_ref, as_ref, ad_ref):
    h = jnp.dot(x_ref[...], w_ref[...], preferred_element_type=jnp.float32)
    h_ref[...] = h
    as_ref[...] = jnp.dot(h * asf_ref[...], g_ref[...],
                          preferred_element_type=jnp.float32)
    ad_ref[...] = jnp.dot(h * adf_ref[...], g_ref[...],
                          preferred_element_type=jnp.float32)


def _edge_body(n_real, src_ref, dst_ref, as_ref, ad_ref, att_ref, den_ref):
    p = pl.program_id(0)
    i = pl.program_id(1)

    @pl.when((p == 0) & (i == 0))
    def _():
        den_ref[...] = jnp.zeros_like(den_ref)

    def _ex(e):
        s = src_ref[0, 0, e]
        d = dst_ref[0, 0, e]
        av = as_ref[pl.ds(s, 1), :]
        bv = ad_ref[pl.ds(d, 1), :]
        alpha = av + bv
        alpha = jnp.where(alpha > 0, alpha, NEG_SLOPE * alpha)
        valid = (i * CH + e) < n_real
        return d, jnp.where(valid, jnp.exp(alpha), jnp.zeros_like(alpha))

    @pl.when(p == 0)
    def _():
        def body(e, _):
            d, ex = _ex(e)
            den_ref[pl.ds(d, 1), :] = den_ref[pl.ds(d, 1), :] + ex
            return _
        jax.lax.fori_loop(0, CH, body, None)

    @pl.when(p == 1)
    def _():
        def body(e, _):
            d, ex = _ex(e)
            den = den_ref[pl.ds(d, 1), :]
            att_ref[0, pl.ds(e, 1), :] = ex / (den + 1e-16)
            return _
        jax.lax.fori_loop(0, CH, body, None)


def _agg_body(nch, src_ref, dst_ref, att_ref, h_ref, bias_ref, out_ref):
    i = pl.program_id(1)

    @pl.when(i == 0)
    def _():
        out_ref[...] = jnp.zeros_like(out_ref)

    def body(e, _):
        s = src_ref[0, 0, e]
        d = dst_ref[0, 0, e]
        a = att_ref[0, 0, 0, e]
        out_ref[pl.ds(d, 1), :] = (out_ref[pl.ds(d, 1), :]
                                   + h_ref[pl.ds(s, 1), :] * a)
        return _
    jax.lax.fori_loop(0, CH, body, None)

    @pl.when(i == nch - 1)
    def _():
        out_ref[...] = out_ref[...] + bias_ref[...]


def kernel(x, edge_index, W, att_src, att_dst, bias):
    N, IN = x.shape
    _, H, C = att_src.shape
    HC = H * C
    E = edge_index.shape[1]
    E2 = E + N
    nch = -(-E2 // CH)
    E2p = nch * CH

    loop = jnp.arange(N, dtype=jnp.int32)
    src = jnp.concatenate([edge_index[0], loop])
    dst = jnp.concatenate([edge_index[1], loop])
    pad = E2p - E2
    src = jnp.pad(src, (0, pad)).reshape(nch, 1, CH)
    dst = jnp.pad(dst, (0, pad)).reshape(nch, 1, CH)

    g = (jnp.arange(HC) // C)[:, None] == jnp.arange(H)[None, :]
    g = g.astype(jnp.float32)
    asf = att_src.reshape(1, HC)
    adf = att_dst.reshape(1, HC)

    nb = next(b for b in (400, 256, 128, 64, 32, 16, 8, 4, 2, 1) if N % b == 0)
    h, a_src, a_dst = pl.pallas_call(
        _proj_body,
        grid=(N // nb,),
        in_specs=[
            pl.BlockSpec((nb, IN), lambda i: (i, 0)),
            pl.BlockSpec((IN, HC), lambda i: (0, 0)),
            pl.BlockSpec((1, HC), lambda i: (0, 0)),
            pl.BlockSpec((1, HC), lambda i: (0, 0)),
            pl.BlockSpec((HC, H), lambda i: (0, 0)),
        ],
        out_specs=[
            pl.BlockSpec((nb, HC), lambda i: (i, 0)),
            pl.BlockSpec((nb, H), lambda i: (i, 0)),
            pl.BlockSpec((nb, H), lambda i: (i, 0)),
        ],
        out_shape=[
            jax.ShapeDtypeStruct((N, HC), jnp.float32),
            jax.ShapeDtypeStruct((N, H), jnp.float32),
            jax.ShapeDtypeStruct((N, H), jnp.float32),
        ],
    )(x, W, asf, adf, g)

    att = pl.pallas_call(
        functools.partial(_edge_body, E2),
        grid=(2, nch),
        in_specs=[
            pl.BlockSpec((1, 1, CH), lambda p, i: (i, 0, 0),
                         memory_space=pltpu.SMEM),
            pl.BlockSpec((1, 1, CH), lambda p, i: (i, 0, 0),
                         memory_space=pltpu.SMEM),
            pl.BlockSpec((N, H), lambda p, i: (0, 0)),
            pl.BlockSpec((N, H), lambda p, i: (0, 0)),
        ],
        out_specs=pl.BlockSpec((1, CH, H), lambda p, i: (p, i, 0)),
        out_shape=jax.ShapeDtypeStruct((2, E2p, H), jnp.float32),
        scratch_shapes=[
            pltpu.VMEM((N, H), jnp.float32),
        ],
        compiler_params=pltpu.CompilerParams(
            dimension_semantics=("arbitrary", "arbitrary")),
    )(src, dst, a_src, a_dst)

    att_t = att[1].T.reshape(H, nch, 1, CH)
    bias2 = bias.reshape(1, HC)

    out = pl.pallas_call(
        functools.partial(_agg_body, nch),
        grid=(H, nch),
        in_specs=[
            pl.BlockSpec((1, 1, CH), lambda hh, i: (i, 0, 0),
                         memory_space=pltpu.SMEM),
            pl.BlockSpec((1, 1, CH), lambda hh, i: (i, 0, 0),
                         memory_space=pltpu.SMEM),
            pl.BlockSpec((1, 1, 1, CH), lambda hh, i: (hh, i, 0, 0),
                         memory_space=pltpu.SMEM),
            pl.BlockSpec((N, C), lambda hh, i: (0, hh)),
            pl.BlockSpec((1, C), lambda hh, i: (0, hh)),
        ],
        out_specs=pl.BlockSpec((N, C), lambda hh, i: (0, hh)),
        out_shape=jax.ShapeDtypeStruct((N, HC), jnp.float32),
        compiler_params=pltpu.CompilerParams(
            dimension_semantics=("arbitrary", "arbitrary")),
    )(src, dst, att_t, h, bias2)
    return out
